# baseline (speedup 1.0000x reference)
import functools

import jax
import jax.numpy as jnp
from jax import lax
from jax.experimental import pallas as pl
from jax.experimental.pallas import tpu as pltpu
from jax.experimental.pallas import tpu_sc as plsc

NUM_EMB = 1000000
DIM = 64
B_TOTAL = 4096 * 50
NC = 2
NS = 16
NW = NC * NS
B_PER_W = B_TOTAL // NW
CHUNK = 128
N_CHUNKS = B_PER_W // CHUNK


def _make_kernel():
    mesh = plsc.VectorSubcoreMesh(core_axis_name="c", subcore_axis_name="s")

    @functools.partial(
        pl.kernel,
        out_type=jax.ShapeDtypeStruct((B_TOTAL, DIM), jnp.float32),
        mesh=mesh,
        scratch_types=[
            pltpu.VMEM((N_CHUNKS, CHUNK), jnp.int32),
            pltpu.VMEM((CHUNK, DIM), jnp.float32),
            pltpu.SemaphoreType.DMA,
        ],
        compiler_params=pltpu.CompilerParams(use_tc_tiling_on_sc=False),
    )
    def emb_kernel(tok_hbm, emb_hbm, out_hbm, idx_v, rows_v, sem):
        wid = lax.axis_index("s") * NC + lax.axis_index("c")
        base = wid * B_PER_W
        pltpu.sync_copy(tok_hbm.at[wid], idx_v)

        def chunk_body(j, carry):
            pltpu.async_copy(emb_hbm.at[idx_v.at[j]], rows_v, sem).wait()
            pltpu.sync_copy(rows_v, out_hbm.at[pl.ds(base + j * CHUNK, CHUNK)])
            return carry

        lax.fori_loop(0, N_CHUNKS, chunk_body, 0, unroll=False)

    return emb_kernel


_emb_kernel = _make_kernel()


def kernel(token_ids, embeddings):
    ids = token_ids.reshape(NW, N_CHUNKS, CHUNK).astype(jnp.int32)
    out = _emb_kernel(ids, embeddings)
    return out.reshape(token_ids.shape[0], token_ids.shape[1], DIM)


# speedup vs baseline: 1.0464x; 1.0464x over previous
import functools

import jax
import jax.numpy as jnp
from jax import lax
from jax.experimental import pallas as pl
from jax.experimental.pallas import tpu as pltpu
from jax.experimental.pallas import tpu_sc as plsc

NUM_EMB = 1000000
DIM = 64
B_TOTAL = 4096 * 50
NC = 2
NS = 16
NW = NC * NS
B_PER_W = B_TOTAL // NW
CHUNK = 128
N_CHUNKS = B_PER_W // CHUNK
NBUF = 8
PF = NBUF - 1
N_MAIN = (N_CHUNKS // NBUF) * NBUF


def _make_kernel():
    mesh = plsc.VectorSubcoreMesh(core_axis_name="c", subcore_axis_name="s")

    @functools.partial(
        pl.kernel,
        out_type=jax.ShapeDtypeStruct((B_TOTAL, DIM), jnp.float32),
        mesh=mesh,
        scratch_types=[
            pltpu.VMEM((N_CHUNKS, CHUNK), jnp.int32),
            pltpu.VMEM((NBUF, CHUNK, DIM), jnp.float32),
        ] + [pltpu.SemaphoreType.DMA] * NBUF,
        compiler_params=pltpu.CompilerParams(use_tc_tiling_on_sc=False),
    )
    def emb_kernel(tok_hbm, emb_hbm, out_hbm, idx_v, rows_v, *sems):
        wid = lax.axis_index("s") * NC + lax.axis_index("c")
        base = wid * B_PER_W
        pltpu.sync_copy(tok_hbm.at[wid], idx_v)

        def gather(j, b):
            pltpu.async_copy(emb_hbm.at[idx_v.at[j]], rows_v.at[b], sems[b])

        def drain(b):
            pltpu.make_async_copy(
                emb_hbm.at[pl.ds(0, CHUNK)], rows_v.at[b], sems[b]).wait()

        def put(j, b):
            pltpu.sync_copy(
                rows_v.at[b], out_hbm.at[pl.ds(base + j * CHUNK, CHUNK)])

        for c in range(PF):
            gather(c, c)

        def group(g0, carry):
            g = g0 * NBUF
            for b in range(NBUF):
                j = g + b
                drain(b)

                @pl.when(j + PF < N_CHUNKS)
                def _():
                    gather(j + PF, (b + PF) % NBUF)

                put(j, b)
            return carry

        lax.fori_loop(0, N_MAIN // NBUF, group, 0, unroll=False)

        for j in range(N_MAIN, N_CHUNKS):
            b = j % NBUF
            drain(b)
            put(j, b)

    return emb_kernel


_emb_kernel = _make_kernel()


def kernel(token_ids, embeddings):
    ids = token_ids.reshape(NW, N_CHUNKS, CHUNK).astype(jnp.int32)
    out = _emb_kernel(ids, embeddings)
    return out.reshape(token_ids.shape[0], token_ids.shape[1], DIM)


# speedup vs baseline: 1.0467x; 1.0003x over previous
import functools

import jax
import jax.numpy as jnp
from jax import lax
from jax.experimental import pallas as pl
from jax.experimental.pallas import tpu as pltpu
from jax.experimental.pallas import tpu_sc as plsc

NUM_EMB = 1000000
DIM = 64
N_TOK = 4096
SEQ = 50
NC = 2
NS = 16
NW = NC * NS
ROWS_PER_W = N_TOK // NW
NBUF = 8
PF = NBUF - 1
N_MAIN = (ROWS_PER_W // NBUF) * NBUF


def _make_kernel():
    mesh = plsc.VectorSubcoreMesh(core_axis_name="c", subcore_axis_name="s")

    @functools.partial(
        pl.kernel,
        out_type=jax.ShapeDtypeStruct((N_TOK, SEQ, DIM), jnp.float32),
        mesh=mesh,
        scratch_types=[
            pltpu.VMEM((ROWS_PER_W, SEQ), jnp.int32),
            pltpu.VMEM((NBUF, SEQ, DIM), jnp.float32),
        ] + [pltpu.SemaphoreType.DMA] * NBUF,
        compiler_params=pltpu.CompilerParams(use_tc_tiling_on_sc=False),
    )
    def emb_kernel(tok_hbm, emb_hbm, out_hbm, idx_v, rows_v, *sems):
        wid = lax.axis_index("s") * NC + lax.axis_index("c")
        base = wid * ROWS_PER_W
        pltpu.sync_copy(tok_hbm.at[pl.ds(base, ROWS_PER_W)], idx_v)

        def gather(r, b):
            pltpu.async_copy(emb_hbm.at[idx_v.at[r]], rows_v.at[b], sems[b])

        def drain(b):
            pltpu.make_async_copy(
                emb_hbm.at[pl.ds(0, SEQ)], rows_v.at[b], sems[b]).wait()

        def put(r, b):
            pltpu.sync_copy(rows_v.at[b], out_hbm.at[base + r])

        for c in range(PF):
            gather(c, c)

        def group(g0, carry):
            g = g0 * NBUF
            for b in range(NBUF):
                r = g + b
                drain(b)

                @pl.when(r + PF < ROWS_PER_W)
                def _():
                    gather(r + PF, (b + PF) % NBUF)

                put(r, b)
            return carry

        lax.fori_loop(0, N_MAIN // NBUF, group, 0, unroll=False)

    return emb_kernel


_emb_kernel = _make_kernel()


def kernel(token_ids, embeddings):
    return _emb_kernel(token_ids.astype(jnp.int32), embeddings)
